# Initial kernel scaffold; baseline (speedup 1.0000x reference)
import jax
import jax.numpy as jnp
from jax import lax
from jax.experimental import pallas as pl
from jax.experimental.pallas import tpu as pltpu
from jax.experimental.pallas import tpu_sc as plsc

N = 10000
E = 160000
H = 256
HH = 128
NSUB = 16
NCORE = 2
BLK = 128
NBLK = 80
EPW = NBLK * BLK
E_PAD = NSUB * EPW
NROWS = N + 16
RPS = N // NSUB
NLANE = 16


def _sc_layer_body(x0, x1, src, dst, av, wv, bv, h0, h1,
                   src_v, dst_v, a_sm, w_v, b_v, rows_v, msg_v, aggr_sp,
                   gsem):
    cid = lax.axis_index("c")
    sid = lax.axis_index("s")

    pltpu.sync_copy(wv.at[cid], w_v)
    pltpu.sync_copy(bv.at[cid], b_v)
    pltpu.sync_copy(src.at[sid], src_v)
    pltpu.sync_copy(dst.at[sid], dst_v)

    r0 = sid * RPS

    def init(x_hbm):
        pltpu.sync_copy(x_hbm.at[pl.ds(r0, RPS)], aggr_sp.at[pl.ds(r0, RPS)])

    def process(x_hbm):
        @pl.loop(0, NBLK)
        def _blk(b):
            pltpu.sync_copy(av.at[sid, b], a_sm)
            pltpu.async_copy(x_hbm.at[src_v.at[b]], rows_v, gsem).wait()

            @pl.loop(0, BLK)
            def _edge(j):
                a = a_sm[j]
                for c in range(HH // NLANE):
                    sl = pl.ds(c * NLANE, NLANE)
                    msg_v[j, sl] = jnp.maximum(
                        rows_v[j, sl] + a * w_v[sl] + b_v[sl], 0.0)

            pltpu.sync_copy(msg_v, aggr_sp.at[dst_v.at[b]], add=True)

    def writeout(h_hbm):
        pltpu.sync_copy(aggr_sp.at[pl.ds(r0, RPS)], h_hbm.at[pl.ds(r0, RPS)])

    @pl.when(cid == 0)
    def _():
        init(x0)

    @pl.when(cid == 1)
    def _():
        init(x1)

    plsc.subcore_barrier()

    @pl.when(cid == 0)
    def _():
        process(x0)

    @pl.when(cid == 1)
    def _():
        process(x1)

    plsc.subcore_barrier()

    @pl.when(cid == 0)
    def _():
        writeout(h0)

    @pl.when(cid == 1)
    def _():
        writeout(h1)


def _gine_layer_sc(x0, x1, src_r, dst_r, a_r, wv, bv):
    out = jax.ShapeDtypeStruct((N, HH), jnp.float32)
    return pl.kernel(
        _sc_layer_body,
        out_type=(out, out),
        mesh=plsc.VectorSubcoreMesh(core_axis_name="c", subcore_axis_name="s"),
        scratch_types=[
            pltpu.VMEM((NBLK, BLK), jnp.int32),
            pltpu.VMEM((NBLK, BLK), jnp.int32),
            pltpu.SMEM((BLK,), jnp.float32),
            pltpu.VMEM((HH,), jnp.float32),
            pltpu.VMEM((HH,), jnp.float32),
            pltpu.VMEM((BLK, HH), jnp.float32),
            pltpu.VMEM((BLK, HH), jnp.float32),
            pltpu.VMEM_SHARED((NROWS, HH), jnp.float32),
            pltpu.SemaphoreType.DMA,
        ],
    )(x0, x1, src_r, dst_r, a_r, wv, bv)


RB = 1000


def _mlp_body(h0_ref, h1_ref, w1t_ref, b1_ref, w2t_ref, b2_ref,
              y0_ref, y1_ref):
    h = jnp.concatenate([h0_ref[...], h1_ref[...]], axis=1)
    z = jnp.maximum(
        jnp.dot(h, w1t_ref[...], preferred_element_type=jnp.float32)
        + b1_ref[...], 0.0)
    y = (jnp.dot(z, w2t_ref[...], preferred_element_type=jnp.float32)
         + b2_ref[...])
    y0_ref[...] = y[:, :HH]
    y1_ref[...] = y[:, HH:]


def _mlp(h0, h1, w1t, b1r, w2t, b2r):
    return pl.pallas_call(
        _mlp_body,
        grid=(N // RB,),
        in_specs=[
            pl.BlockSpec((RB, HH), lambda i: (i, 0)),
            pl.BlockSpec((RB, HH), lambda i: (i, 0)),
            pl.BlockSpec((H, H), lambda i: (0, 0)),
            pl.BlockSpec((1, H), lambda i: (0, 0)),
            pl.BlockSpec((H, H), lambda i: (0, 0)),
            pl.BlockSpec((1, H), lambda i: (0, 0)),
        ],
        out_specs=[pl.BlockSpec((RB, HH), lambda i: (i, 0)),
                   pl.BlockSpec((RB, HH), lambda i: (i, 0))],
        out_shape=[jax.ShapeDtypeStruct((N, HH), jnp.float32)] * 2,
    )(h0, h1, w1t, b1r, w2t, b2r)


def _final_body(h0_ref, h1_ref, w1t_ref, b1_ref, w2t_ref, b2_ref,
                wo_ref, bo_ref, o_ref):
    i = pl.program_id(0)
    h = jnp.concatenate([h0_ref[...], h1_ref[...]], axis=1)
    z = jnp.maximum(
        jnp.dot(h, w1t_ref[...], preferred_element_type=jnp.float32)
        + b1_ref[...], 0.0)
    y = (jnp.dot(z, w2t_ref[...], preferred_element_type=jnp.float32)
         + b2_ref[...])
    part = jnp.sum(y * wo_ref[...])
    prev = jnp.where(i == 0, bo_ref[0, 0], o_ref[0, 0])
    o_ref[0, 0] = prev + part


def _final(h0, h1, w1t, b1r, w2t, b2r, wor, bor):
    return pl.pallas_call(
        _final_body,
        grid=(N // RB,),
        in_specs=[
            pl.BlockSpec((RB, HH), lambda i: (i, 0)),
            pl.BlockSpec((RB, HH), lambda i: (i, 0)),
            pl.BlockSpec((H, H), lambda i: (0, 0)),
            pl.BlockSpec((1, H), lambda i: (0, 0)),
            pl.BlockSpec((H, H), lambda i: (0, 0)),
            pl.BlockSpec((1, H), lambda i: (0, 0)),
            pl.BlockSpec((1, H), lambda i: (0, 0)),
            pl.BlockSpec(memory_space=pltpu.SMEM),
        ],
        out_specs=pl.BlockSpec(memory_space=pltpu.SMEM),
        out_shape=jax.ShapeDtypeStruct((1, 1), jnp.float32),
    )(h0, h1, w1t, b1r, w2t, b2r, wor, bor)


def kernel(X, edge_index, edge_attr, We_w, We_b, W1, b1, W2, b2, Wo, bo):
    src = edge_index[0].astype(jnp.int32)
    dst = edge_index[1].astype(jnp.int32)
    npad = E_PAD - E
    pidx = jnp.arange(npad, dtype=jnp.int32)
    src_r = jnp.concatenate([src, pidx % 8]).reshape(NSUB, NBLK, BLK)
    dst_r = jnp.concatenate([dst, N + (pidx % 8)]).reshape(NSUB, NBLK, BLK)
    a_r = jnp.concatenate(
        [edge_attr[:, 0], jnp.zeros((npad,), jnp.float32)]
    ).reshape(NSUB, NBLK, BLK)
    wv = We_w[:, 0].reshape(NCORE, HH)
    bv = We_b.reshape(NCORE, HH)
    w1t = W1.T
    b1r = b1.reshape(1, H)
    w2t = W2.T
    b2r = b2.reshape(1, H)
    wor = Wo.reshape(1, H)
    bor = bo.reshape(1, 1)

    x0 = X[:, :HH]
    x1 = X[:, HH:]
    h0 = h1 = None
    for layer in range(3):
        h0, h1 = _gine_layer_sc(x0, x1, src_r, dst_r, a_r, wv, bv)
        if layer < 2:
            x0, x1 = _mlp(h0, h1, w1t, b1r, w2t, b2r)
    prop = _final(h0, h1, w1t, b1r, w2t, b2r, wor, bor)
    return prop.reshape(1)


# speedup vs baseline: 1.4448x; 1.4448x over previous
import jax
import jax.numpy as jnp
from jax import lax
from jax.experimental import pallas as pl
from jax.experimental.pallas import tpu as pltpu
from jax.experimental.pallas import tpu_sc as plsc

N = 10000
E = 160000
H = 256
HH = 128
NSUB = 16
NCORE = 2
BLK = 128
NBLK = 80
ICH = 16
EPW = NBLK * BLK
E_PAD = NSUB * EPW
NROWS = N + 16
RPS = 624
RPS_LAST = N - RPS * (NSUB - 1)
NLANE = 16


def _sc_layer_body(x0, x1, src, dst, av, wv, bv, h0, h1,
                   src_v, dst_v, a_v, w_v, b_v, rows_v, msg_v, aggr_sp,
                   gsem):
    cid = lax.axis_index("c")
    sid = lax.axis_index("s")

    pltpu.sync_copy(wv.at[cid], w_v)
    pltpu.sync_copy(bv.at[cid], b_v)

    r0 = sid * RPS

    def init(x_hbm):
        @pl.when(sid < NSUB - 1)
        def _():
            pltpu.sync_copy(x_hbm.at[pl.ds(r0, RPS)],
                            aggr_sp.at[pl.ds(r0, RPS)])

        @pl.when(sid == NSUB - 1)
        def _():
            pltpu.sync_copy(x_hbm.at[pl.ds(RPS * (NSUB - 1), RPS_LAST)],
                            aggr_sp.at[pl.ds(RPS * (NSUB - 1), RPS_LAST)])

    def process(x_hbm):
        @pl.loop(0, NBLK // ICH)
        def _chunk(cc):
            pltpu.sync_copy(src.at[sid, pl.ds(cc * ICH, ICH)], src_v)
            pltpu.sync_copy(dst.at[sid, pl.ds(cc * ICH, ICH)], dst_v)
            pltpu.sync_copy(av.at[sid, pl.ds(cc * ICH, ICH)], a_v)

            @pl.loop(0, ICH)
            def _blk(b):
                pltpu.async_copy(x_hbm.at[src_v.at[b]], rows_v, gsem).wait()

                @pl.loop(0, BLK, step=NLANE)
                def _grp(j0):
                    a16 = a_v[b, pl.ds(j0, NLANE)]
                    for jj in range(NLANE):
                        a = a16[jj]
                        for c in range(HH // NLANE):
                            sl = pl.ds(c * NLANE, NLANE)
                            msg_v[j0 + jj, sl] = jnp.maximum(
                                rows_v[j0 + jj, sl] + a * w_v[sl] + b_v[sl],
                                0.0)

                pltpu.sync_copy(msg_v, aggr_sp.at[dst_v.at[b]], add=True)

    def writeout(h_hbm):
        @pl.when(sid < NSUB - 1)
        def _():
            pltpu.sync_copy(aggr_sp.at[pl.ds(r0, RPS)],
                            h_hbm.at[pl.ds(r0, RPS)])

        @pl.when(sid == NSUB - 1)
        def _():
            pltpu.sync_copy(aggr_sp.at[pl.ds(RPS * (NSUB - 1), RPS_LAST)],
                            h_hbm.at[pl.ds(RPS * (NSUB - 1), RPS_LAST)])

    @pl.when(cid == 0)
    def _():
        init(x0)

    @pl.when(cid == 1)
    def _():
        init(x1)

    plsc.subcore_barrier()

    @pl.when(cid == 0)
    def _():
        process(x0)

    @pl.when(cid == 1)
    def _():
        process(x1)

    plsc.subcore_barrier()

    @pl.when(cid == 0)
    def _():
        writeout(h0)

    @pl.when(cid == 1)
    def _():
        writeout(h1)


def _gine_layer_sc(x0, x1, src_r, dst_r, a_r, wv, bv):
    out = jax.ShapeDtypeStruct((N, HH), jnp.float32)
    return pl.kernel(
        _sc_layer_body,
        out_type=(out, out),
        mesh=plsc.VectorSubcoreMesh(core_axis_name="c", subcore_axis_name="s"),
        scratch_types=[
            pltpu.VMEM((ICH, BLK), jnp.int32),
            pltpu.VMEM((ICH, BLK), jnp.int32),
            pltpu.VMEM((ICH, BLK), jnp.float32),
            pltpu.VMEM((HH,), jnp.float32),
            pltpu.VMEM((HH,), jnp.float32),
            pltpu.VMEM((BLK, HH), jnp.float32),
            pltpu.VMEM((BLK, HH), jnp.float32),
            pltpu.VMEM_SHARED((NROWS, HH), jnp.float32),
            pltpu.SemaphoreType.DMA,
        ],
    )(x0, x1, src_r, dst_r, a_r, wv, bv)


RB = 1000


def _mlp_body(h0_ref, h1_ref, w1t_ref, b1_ref, w2t_ref, b2_ref,
              y0_ref, y1_ref):
    h = jnp.concatenate([h0_ref[...], h1_ref[...]], axis=1)
    z = jnp.maximum(
        jnp.dot(h, w1t_ref[...], preferred_element_type=jnp.float32)
        + b1_ref[...], 0.0)
    y = (jnp.dot(z, w2t_ref[...], preferred_element_type=jnp.float32)
         + b2_ref[...])
    y0_ref[...] = y[:, :HH]
    y1_ref[...] = y[:, HH:]


def _mlp(h0, h1, w1t, b1r, w2t, b2r):
    return pl.pallas_call(
        _mlp_body,
        grid=(N // RB,),
        in_specs=[
            pl.BlockSpec((RB, HH), lambda i: (i, 0)),
            pl.BlockSpec((RB, HH), lambda i: (i, 0)),
            pl.BlockSpec((H, H), lambda i: (0, 0)),
            pl.BlockSpec((1, H), lambda i: (0, 0)),
            pl.BlockSpec((H, H), lambda i: (0, 0)),
            pl.BlockSpec((1, H), lambda i: (0, 0)),
        ],
        out_specs=[pl.BlockSpec((RB, HH), lambda i: (i, 0)),
                   pl.BlockSpec((RB, HH), lambda i: (i, 0))],
        out_shape=[jax.ShapeDtypeStruct((N, HH), jnp.float32)] * 2,
    )(h0, h1, w1t, b1r, w2t, b2r)


def _final_body(h0_ref, h1_ref, w1t_ref, b1_ref, w2t_ref, b2_ref,
                wo_ref, bo_ref, o_ref):
    i = pl.program_id(0)
    h = jnp.concatenate([h0_ref[...], h1_ref[...]], axis=1)
    z = jnp.maximum(
        jnp.dot(h, w1t_ref[...], preferred_element_type=jnp.float32)
        + b1_ref[...], 0.0)
    y = (jnp.dot(z, w2t_ref[...], preferred_element_type=jnp.float32)
         + b2_ref[...])
    part = jnp.sum(y * wo_ref[...])
    prev = jnp.where(i == 0, bo_ref[0, 0], o_ref[0, 0])
    o_ref[0, 0] = prev + part


def _final(h0, h1, w1t, b1r, w2t, b2r, wor, bor):
    return pl.pallas_call(
        _final_body,
        grid=(N // RB,),
        in_specs=[
            pl.BlockSpec((RB, HH), lambda i: (i, 0)),
            pl.BlockSpec((RB, HH), lambda i: (i, 0)),
            pl.BlockSpec((H, H), lambda i: (0, 0)),
            pl.BlockSpec((1, H), lambda i: (0, 0)),
            pl.BlockSpec((H, H), lambda i: (0, 0)),
            pl.BlockSpec((1, H), lambda i: (0, 0)),
            pl.BlockSpec((1, H), lambda i: (0, 0)),
            pl.BlockSpec(memory_space=pltpu.SMEM),
        ],
        out_specs=pl.BlockSpec(memory_space=pltpu.SMEM),
        out_shape=jax.ShapeDtypeStruct((1, 1), jnp.float32),
    )(h0, h1, w1t, b1r, w2t, b2r, wor, bor)


def kernel(X, edge_index, edge_attr, We_w, We_b, W1, b1, W2, b2, Wo, bo):
    src = edge_index[0].astype(jnp.int32)
    dst = edge_index[1].astype(jnp.int32)
    npad = E_PAD - E
    pidx = jnp.arange(npad, dtype=jnp.int32)
    src_r = jnp.concatenate([src, pidx % 8]).reshape(NSUB, NBLK, BLK)
    dst_r = jnp.concatenate([dst, N + (pidx % 8)]).reshape(NSUB, NBLK, BLK)
    a_r = jnp.concatenate(
        [edge_attr[:, 0], jnp.zeros((npad,), jnp.float32)]
    ).reshape(NSUB, NBLK, BLK)
    wv = We_w[:, 0].reshape(NCORE, HH)
    bv = We_b.reshape(NCORE, HH)
    w1t = W1.T
    b1r = b1.reshape(1, H)
    w2t = W2.T
    b2r = b2.reshape(1, H)
    wor = Wo.reshape(1, H)
    bor = bo.reshape(1, 1)

    x0 = X[:, :HH]
    x1 = X[:, HH:]
    h0 = h1 = None
    for layer in range(3):
        h0, h1 = _gine_layer_sc(x0, x1, src_r, dst_r, a_r, wv, bv)
        if layer < 2:
            x0, x1 = _mlp(h0, h1, w1t, b1r, w2t, b2r)
    prop = _final(h0, h1, w1t, b1r, w2t, b2r, wor, bor)
    return prop.reshape(1)


# speedup vs baseline: 6.3103x; 4.3676x over previous
import dataclasses

import jax
import jax.numpy as jnp
from jax import lax
from jax.experimental import pallas as pl
from jax.experimental.pallas import tpu as pltpu
from jax.experimental.pallas import tpu_sc as plsc

N = 10000
E = 160000
H = 256
HH = 128
NSUB = 16
NCORE = 2
BLK = 128
NBLK = 80
ICH = 16
EPW = NBLK * BLK
E_PAD = NSUB * EPW
NROWS = N + 16
RPS = 624
RPS_LAST = N - RPS * (NSUB - 1)
NLANE = 16


def _sc_layer_body(x0, x1, src, dst, av, wv, bv, h0, h1,
                   src_v, dst_v, a_v, w_v, b_v, rows0, rows1, aggr_sp,
                   gsem0, gsem1):
    cid = lax.axis_index("c")
    sid = lax.axis_index("s")

    pltpu.sync_copy(wv.at[cid], w_v)
    pltpu.sync_copy(bv.at[cid], b_v)
    ws = [w_v[pl.ds(c * NLANE, NLANE)] for c in range(HH // NLANE)]
    bs = [b_v[pl.ds(c * NLANE, NLANE)] for c in range(HH // NLANE)]

    r0 = sid * RPS

    def init(x_hbm):
        @pl.when(sid < NSUB - 1)
        def _():
            pltpu.sync_copy(x_hbm.at[pl.ds(r0, RPS)],
                            aggr_sp.at[pl.ds(r0, RPS)])

        @pl.when(sid == NSUB - 1)
        def _():
            pltpu.sync_copy(x_hbm.at[pl.ds(RPS * (NSUB - 1), RPS_LAST)],
                            aggr_sp.at[pl.ds(RPS * (NSUB - 1), RPS_LAST)])

    def compute(rb, b):
        @plsc.parallel_loop(0, BLK, step=1, unroll=2)
        def _edge(j):
            jv = jnp.full((NLANE,), j, dtype=jnp.int32)
            a16 = plsc.load_gather(a_v.at[b], [jv])
            for c in range(HH // NLANE):
                sl = pl.ds(c * NLANE, NLANE)
                rb[j, sl] = jnp.maximum(
                    rb[j, sl] + a16 * ws[c] + bs[c], 0.0)

    def process(x_hbm):
        @pl.loop(0, NBLK // ICH)
        def _chunk(cc):
            pltpu.sync_copy(src.at[sid, pl.ds(cc * ICH, ICH)], src_v)
            pltpu.sync_copy(dst.at[sid, pl.ds(cc * ICH, ICH)], dst_v)
            pltpu.sync_copy(av.at[sid, pl.ds(cc * ICH, ICH)], a_v)
            pltpu.async_copy(x_hbm.at[src_v.at[0]], rows0, gsem0)

            @pl.loop(0, ICH, step=2)
            def _pair(b):
                pltpu.make_async_copy(
                    x_hbm.at[src_v.at[b]], rows0, gsem0).wait()
                pltpu.async_copy(x_hbm.at[src_v.at[b + 1]], rows1, gsem1)
                compute(rows0, b)
                pltpu.sync_copy(rows0, aggr_sp.at[dst_v.at[b]], add=True)

                pltpu.make_async_copy(
                    x_hbm.at[src_v.at[b + 1]], rows1, gsem1).wait()

                @pl.when(b + 2 < ICH)
                def _():
                    pltpu.async_copy(x_hbm.at[src_v.at[b + 2]], rows0, gsem0)

                compute(rows1, b + 1)
                pltpu.sync_copy(rows1, aggr_sp.at[dst_v.at[b + 1]], add=True)

    def writeout(h_hbm):
        @pl.when(sid < NSUB - 1)
        def _():
            pltpu.sync_copy(aggr_sp.at[pl.ds(r0, RPS)],
                            h_hbm.at[pl.ds(r0, RPS)])

        @pl.when(sid == NSUB - 1)
        def _():
            pltpu.sync_copy(aggr_sp.at[pl.ds(RPS * (NSUB - 1), RPS_LAST)],
                            h_hbm.at[pl.ds(RPS * (NSUB - 1), RPS_LAST)])

    @pl.when(cid == 0)
    def _():
        init(x0)

    @pl.when(cid == 1)
    def _():
        init(x1)

    plsc.subcore_barrier()

    @pl.when(cid == 0)
    def _():
        process(x0)

    @pl.when(cid == 1)
    def _():
        process(x1)

    plsc.subcore_barrier()

    @pl.when(cid == 0)
    def _():
        writeout(h0)

    @pl.when(cid == 1)
    def _():
        writeout(h1)


def _sc_compiler_params():
    cp = pltpu.CompilerParams()
    if "needs_layout_passes" in pltpu.CompilerParams.__dataclass_fields__:
        cp = dataclasses.replace(cp, needs_layout_passes=False)
    return cp


def _gine_layer_sc(x0, x1, src_r, dst_r, a_r, wv, bv):
    out = jax.ShapeDtypeStruct((N, HH), jnp.float32)
    return pl.kernel(
        _sc_layer_body,
        out_type=(out, out),
        mesh=plsc.VectorSubcoreMesh(core_axis_name="c", subcore_axis_name="s"),
        compiler_params=_sc_compiler_params(),
        scratch_types=[
            pltpu.VMEM((ICH, BLK), jnp.int32),
            pltpu.VMEM((ICH, BLK), jnp.int32),
            pltpu.VMEM((ICH, BLK), jnp.float32),
            pltpu.VMEM((HH,), jnp.float32),
            pltpu.VMEM((HH,), jnp.float32),
            pltpu.VMEM((BLK, HH), jnp.float32),
            pltpu.VMEM((BLK, HH), jnp.float32),
            pltpu.VMEM_SHARED((NROWS, HH), jnp.float32),
            pltpu.SemaphoreType.DMA,
            pltpu.SemaphoreType.DMA,
        ],
    )(x0, x1, src_r, dst_r, a_r, wv, bv)


RB = 1000


def _mlp_body(h0_ref, h1_ref, w1t_ref, b1_ref, w2t_ref, b2_ref,
              y0_ref, y1_ref):
    h = jnp.concatenate([h0_ref[...], h1_ref[...]], axis=1)
    z = jnp.maximum(
        jnp.dot(h, w1t_ref[...], preferred_element_type=jnp.float32)
        + b1_ref[...], 0.0)
    y = (jnp.dot(z, w2t_ref[...], preferred_element_type=jnp.float32)
         + b2_ref[...])
    y0_ref[...] = y[:, :HH]
    y1_ref[...] = y[:, HH:]


def _mlp(h0, h1, w1t, b1r, w2t, b2r):
    return pl.pallas_call(
        _mlp_body,
        grid=(N // RB,),
        in_specs=[
            pl.BlockSpec((RB, HH), lambda i: (i, 0)),
            pl.BlockSpec((RB, HH), lambda i: (i, 0)),
            pl.BlockSpec((H, H), lambda i: (0, 0)),
            pl.BlockSpec((1, H), lambda i: (0, 0)),
            pl.BlockSpec((H, H), lambda i: (0, 0)),
            pl.BlockSpec((1, H), lambda i: (0, 0)),
        ],
        out_specs=[pl.BlockSpec((RB, HH), lambda i: (i, 0)),
                   pl.BlockSpec((RB, HH), lambda i: (i, 0))],
        out_shape=[jax.ShapeDtypeStruct((N, HH), jnp.float32)] * 2,
    )(h0, h1, w1t, b1r, w2t, b2r)


def _final_body(h0_ref, h1_ref, w1t_ref, b1_ref, w2t_ref, b2_ref,
                wo_ref, bo_ref, o_ref):
    i = pl.program_id(0)
    h = jnp.concatenate([h0_ref[...], h1_ref[...]], axis=1)
    z = jnp.maximum(
        jnp.dot(h, w1t_ref[...], preferred_element_type=jnp.float32)
        + b1_ref[...], 0.0)
    y = (jnp.dot(z, w2t_ref[...], preferred_element_type=jnp.float32)
         + b2_ref[...])
    part = jnp.sum(y * wo_ref[...])
    prev = jnp.where(i == 0, bo_ref[0, 0], o_ref[0, 0])
    o_ref[0, 0] = prev + part


def _final(h0, h1, w1t, b1r, w2t, b2r, wor, bor):
    return pl.pallas_call(
        _final_body,
        grid=(N // RB,),
        in_specs=[
            pl.BlockSpec((RB, HH), lambda i: (i, 0)),
            pl.BlockSpec((RB, HH), lambda i: (i, 0)),
            pl.BlockSpec((H, H), lambda i: (0, 0)),
            pl.BlockSpec((1, H), lambda i: (0, 0)),
            pl.BlockSpec((H, H), lambda i: (0, 0)),
            pl.BlockSpec((1, H), lambda i: (0, 0)),
            pl.BlockSpec((1, H), lambda i: (0, 0)),
            pl.BlockSpec(memory_space=pltpu.SMEM),
        ],
        out_specs=pl.BlockSpec(memory_space=pltpu.SMEM),
        out_shape=jax.ShapeDtypeStruct((1, 1), jnp.float32),
    )(h0, h1, w1t, b1r, w2t, b2r, wor, bor)


def kernel(X, edge_index, edge_attr, We_w, We_b, W1, b1, W2, b2, Wo, bo):
    src = edge_index[0].astype(jnp.int32)
    dst = edge_index[1].astype(jnp.int32)
    npad = E_PAD - E
    pidx = jnp.arange(npad, dtype=jnp.int32)
    src_r = jnp.concatenate([src, pidx % 8]).reshape(NSUB, NBLK, BLK)
    dst_r = jnp.concatenate([dst, N + (pidx % 8)]).reshape(NSUB, NBLK, BLK)
    a_r = jnp.concatenate(
        [edge_attr[:, 0], jnp.zeros((npad,), jnp.float32)]
    ).reshape(NSUB, NBLK, BLK)
    wv = We_w[:, 0].reshape(NCORE, HH)
    bv = We_b.reshape(NCORE, HH)
    w1t = W1.T
    b1r = b1.reshape(1, H)
    w2t = W2.T
    b2r = b2.reshape(1, H)
    wor = Wo.reshape(1, H)
    bor = bo.reshape(1, 1)

    x0 = X[:, :HH]
    x1 = X[:, HH:]
    h0 = h1 = None
    for layer in range(3):
        h0, h1 = _gine_layer_sc(x0, x1, src_r, dst_r, a_r, wv, bv)
        if layer < 2:
            x0, x1 = _mlp(h0, h1, w1t, b1r, w2t, b2r)
    prop = _final(h0, h1, w1t, b1r, w2t, b2r, wor, bor)
    return prop.reshape(1)
